# baseline (speedup 1.0000x reference)
from functools import partial

import jax
import jax.numpy as jnp
from jax import lax
from jax.experimental import pallas as pl
from jax.experimental.pallas import tpu as pltpu
from jax.experimental.pallas import tpu_sc as plsc

_NUM_TILES = 16
_CH = 128


_NB = 3


def _make_sc_agg(N, D, E):
    per_tile = E // _NUM_TILES
    full = per_tile // _CH
    assert full % _NB == 0
    nrounds = full // _NB
    rem_edges = E - _NUM_TILES * full * _CH
    assert rem_edges % _CH == 0
    rem_tiles = rem_edges // _CH
    stripe_pieces = (_CH, _CH, _CH, _CH, 112)
    rows_per_tile = sum(stripe_pieces)
    rem_rows = N - _NUM_TILES * rows_per_tile
    assert rem_rows % 8 == 0 and rem_rows <= _CH

    mesh = plsc.VectorSubcoreMesh(core_axis_name="c", subcore_axis_name="s")

    @partial(
        pl.kernel,
        out_type=(
            jax.ShapeDtypeStruct((N, D), jnp.float32),
            jax.ShapeDtypeStruct((N, D), jnp.float32),
        ),
        mesh=mesh,
        scratch_types=(
            [pltpu.VMEM((2, _CH), jnp.int32) for _ in range(_NB)]
            + [pltpu.VMEM((_CH, D), jnp.float32) for _ in range(_NB)]
            + [pltpu.VMEM_SHARED((N, D), jnp.float32)]
            + [pltpu.SemaphoreType.DMA for _ in range(2 * _NB)]
        ),
    )
    def sc_agg(xr_hbm, er_hbm, xp_hbm, ep_hbm, outr_hbm, outp_hbm, *scratch):
        idxs = scratch[:_NB]
        rows = scratch[_NB:2 * _NB]
        agg_sh = scratch[2 * _NB]
        gsems = scratch[2 * _NB + 1:3 * _NB + 1]
        ssems = scratch[3 * _NB + 1:]
        rows_a = rows[0]
        c = lax.axis_index("c")
        s = lax.axis_index("s")

        def my_stripes(fn, rem):
            off = 0
            for nr in stripe_pieces:
                fn(s * rows_per_tile + off, nr)
                off += nr
            if rem:
                @pl.when(s == 0)
                def _():
                    fn(_NUM_TILES * rows_per_tile, rem)

        @pl.loop(0, _CH)
        def _(r):
            for j in range(D // 16):
                rows_a[r, pl.ds(j * 16, 16)] = jnp.zeros((16,), jnp.float32)

        my_stripes(lambda r0, nr: pltpu.sync_copy(
            rows_a.at[pl.ds(0, nr)], agg_sh.at[pl.ds(r0, nr)]), rem_rows)

        plsc.subcore_barrier()

        def run_graph(x_hbm, e_hbm, out_hbm):
            def g_copy(b):
                return pltpu.make_async_copy(
                    x_hbm.at[idxs[b].at[0]], rows[b], gsems[b])

            def s_copy(b):
                return pltpu.make_async_copy(
                    rows[b], agg_sh.at[idxs[b].at[1]], ssems[b])

            def fetch_idx(base, b):
                pltpu.sync_copy(e_hbm.at[:, pl.ds(base, _CH)], idxs[b])

            ebase = s * (full * _CH)

            @pl.loop(0, nrounds)
            def _(t):
                base = ebase + t * (_NB * _CH)
                for b in range(_NB):
                    @pl.when(t > 0)
                    def _():
                        s_copy(b).wait()
                    fetch_idx(base + b * _CH, b)
                    g_copy(b).start()
                for b in range(_NB):
                    g_copy(b).wait()
                    s_copy(b).start(add=True)

            for b in range(_NB):
                s_copy(b).wait()

            if rem_tiles:
                @pl.when(s < rem_tiles)
                def _():
                    fetch_idx(_NUM_TILES * full * _CH + s * _CH, 0)
                    pltpu.sync_copy(x_hbm.at[idxs[0].at[0]], rows[0])
                    pltpu.sync_copy(rows[0], agg_sh.at[idxs[0].at[1]], add=True)

            plsc.subcore_barrier()

            def writeout(r0, nr):
                pltpu.sync_copy(agg_sh.at[pl.ds(r0, nr)], rows_a.at[pl.ds(0, nr)])
                pltpu.sync_copy(rows_a.at[pl.ds(0, nr)], out_hbm.at[pl.ds(r0, nr)])

            my_stripes(writeout, rem_rows)

        @pl.when(c == 0)
        def _():
            run_graph(xr_hbm, er_hbm, outr_hbm)

        @pl.when(c == 1)
        def _():
            run_graph(xp_hbm, ep_hbm, outp_hbm)

    return sc_agg


def _make_tc(N, D, B, RB):
    nb = N // RB

    def body(x_r_ref, a_r_ref, id_r_ref, x_p_ref, a_p_ref, id_p_ref,
             w1_ref, b1_ref, w2_ref, b2_ref, eps_ref,
             aw0, ab0, aw1, ab1, aw2, ab2, aw3, ab3,
             cw0, cb0, cw1, cb1, cw2, cb2,
             out_a, out_q, feat_ref):
        i = pl.program_id(0)

        @pl.when(i == 0)
        def _():
            feat_ref[...] = jnp.zeros_like(feat_ref)

        w1 = w1_ref[...]
        b1 = b1_ref[...]
        w2 = w2_ref[...]
        b2 = b2_ref[...]
        eps1 = 1.0 + eps_ref[0, 0]
        groups = ((x_r_ref, a_r_ref, id_r_ref), (x_p_ref, a_p_ref, id_p_ref))
        for g, (x_ref, a_ref, id_ref) in enumerate(groups):
            h0 = eps1 * x_ref[...] + a_ref[...]
            h1 = jnp.maximum(
                jnp.dot(h0, w1, preferred_element_type=jnp.float32) + b1, 0.0)
            h2 = jnp.dot(h1, w2, preferred_element_type=jnp.float32) + b2
            ids = id_ref[0]
            oh = (lax.broadcasted_iota(jnp.int32, (B, RB), 0) == ids
                  ).astype(jnp.float32)
            feat_ref[g] += jnp.dot(oh, h2, preferred_element_type=jnp.float32)

        @pl.when(i == nb - 1)
        def _():
            f = feat_ref[...]
            inp = jnp.concatenate([f[0], f[1]], axis=1)
            h = jnp.maximum(
                jnp.dot(inp, aw0[...], preferred_element_type=jnp.float32)
                + ab0[...], 0.0)
            h = jnp.maximum(
                jnp.dot(h, aw1[...], preferred_element_type=jnp.float32)
                + ab1[...], 0.0)
            h = jnp.maximum(
                jnp.dot(h, aw2[...], preferred_element_type=jnp.float32)
                + ab2[...], 0.0)
            out_a[...] = (jnp.dot(h, aw3[...], preferred_element_type=jnp.float32)
                          + ab3[...])
            hc = jnp.maximum(
                jnp.dot(inp, cw0[...], preferred_element_type=jnp.float32)
                + cb0[...], 0.0)
            hc = jnp.maximum(
                jnp.dot(hc, cw1[...], preferred_element_type=jnp.float32)
                + cb1[...], 0.0)
            out_q[...] = (jnp.dot(hc, cw2[...], preferred_element_type=jnp.float32)
                          + cb2[...])

    cst = lambda *shape: pl.BlockSpec(shape, lambda i: tuple(0 for _ in shape))
    blk = pl.BlockSpec((RB, D), lambda i: (i, 0))
    idb = pl.BlockSpec((1, 1, RB), lambda i: (i, 0, 0))

    def run(x_r, a_r, id_r, x_p, a_p, id_p, *ws):
        specs = ([blk, blk, idb, blk, blk, idb]
                 + [cst(*w.shape) for w in ws])
        return pl.pallas_call(
            body,
            grid=(nb,),
            in_specs=specs,
            out_specs=[cst(B, 2 * D), cst(B, 1)],
            out_shape=[
                jax.ShapeDtypeStruct((B, 2 * D), jnp.float32),
                jax.ShapeDtypeStruct((B, 1), jnp.float32),
            ],
            scratch_shapes=[pltpu.VMEM((2, B, D), jnp.float32)],
        )(x_r, a_r, id_r, x_p, a_p, id_p, *ws)

    return run


def kernel(reac_x, reac_edge_index, reac_graph_ids, prod_x, prod_edge_index,
           prod_graph_ids, gin_W1, gin_b1, gin_W2, gin_b2, gin_eps,
           aW0, ab0, aW1, ab1, aW2, ab2, aW3, ab3,
           cW0, cb0, cW1, cb1, cW2, cb2):
    N, D = reac_x.shape
    E = reac_edge_index.shape[1]
    B = 1024
    RB = 5000

    er = reac_edge_index.astype(jnp.int32)
    ep = prod_edge_index.astype(jnp.int32)
    agg_r, agg_p = _make_sc_agg(N, D, E)(reac_x, er, prod_x, ep)

    ids_r = reac_graph_ids.astype(jnp.int32).reshape(N // RB, 1, RB)
    ids_p = prod_graph_ids.astype(jnp.int32).reshape(N // RB, 1, RB)
    actions, q = _make_tc(N, D, B, RB)(
        reac_x, agg_r, ids_r, prod_x, agg_p, ids_p,
        gin_W1, gin_b1.reshape(1, D), gin_W2, gin_b2.reshape(1, D),
        gin_eps.reshape(1, 1),
        aW0, ab0.reshape(1, -1), aW1, ab1.reshape(1, -1),
        aW2, ab2.reshape(1, -1), aW3, ab3.reshape(1, -1),
        cW0, cb0.reshape(1, -1), cW1, cb1.reshape(1, -1),
        cW2, cb2.reshape(1, -1))
    return (actions, q)


# speedup vs baseline: 1.0158x; 1.0158x over previous
from functools import partial

import jax
import jax.numpy as jnp
from jax import lax
from jax.experimental import pallas as pl
from jax.experimental.pallas import tpu as pltpu
from jax.experimental.pallas import tpu_sc as plsc

_NUM_TILES = 16
_CH = 128


_NB = 3


def _make_sc_agg(N, D, E):
    per_tile = E // _NUM_TILES
    full = per_tile // _CH
    assert full % _NB == 0
    nrounds = full // _NB
    rem_edges = E - _NUM_TILES * full * _CH
    assert rem_edges % _CH == 0
    rem_tiles = rem_edges // _CH
    stripe_pieces = (_CH, _CH, _CH, _CH, 112)
    rows_per_tile = sum(stripe_pieces)
    rem_rows = N - _NUM_TILES * rows_per_tile
    assert rem_rows % 8 == 0 and rem_rows <= _CH

    mesh = plsc.VectorSubcoreMesh(core_axis_name="c", subcore_axis_name="s")

    @partial(
        pl.kernel,
        out_type=(
            jax.ShapeDtypeStruct((N, D), jnp.float32),
            jax.ShapeDtypeStruct((N, D), jnp.float32),
        ),
        mesh=mesh,
        scratch_types=(
            [pltpu.VMEM((2, _CH), jnp.int32) for _ in range(_NB)]
            + [pltpu.VMEM((_CH, D), jnp.float32) for _ in range(_NB)]
            + [pltpu.VMEM_SHARED((N, D), jnp.float32)]
            + [pltpu.SemaphoreType.DMA for _ in range(2 * _NB)]
        ),
    )
    def sc_agg(xr_hbm, er_hbm, xp_hbm, ep_hbm, outr_hbm, outp_hbm, *scratch):
        idxs = scratch[:_NB]
        rows = scratch[_NB:2 * _NB]
        agg_sh = scratch[2 * _NB]
        gsems = scratch[2 * _NB + 1:3 * _NB + 1]
        ssems = scratch[3 * _NB + 1:]
        rows_a = rows[0]
        c = lax.axis_index("c")
        s = lax.axis_index("s")

        def my_stripes(fn, rem):
            off = 0
            for nr in stripe_pieces:
                fn(s * rows_per_tile + off, nr)
                off += nr
            if rem:
                @pl.when(s == 0)
                def _():
                    fn(_NUM_TILES * rows_per_tile, rem)

        @pl.loop(0, _CH)
        def _(r):
            for j in range(D // 16):
                rows_a[r, pl.ds(j * 16, 16)] = jnp.zeros((16,), jnp.float32)

        my_stripes(lambda r0, nr: pltpu.sync_copy(
            rows_a.at[pl.ds(0, nr)], agg_sh.at[pl.ds(r0, nr)]), rem_rows)

        plsc.subcore_barrier()

        def run_graph(x_hbm, e_hbm, out_hbm):
            def g_copy(b):
                return pltpu.make_async_copy(
                    x_hbm.at[idxs[b].at[0]], rows[b], gsems[b])

            def s_copy(b):
                return pltpu.make_async_copy(
                    rows[b], agg_sh.at[idxs[b].at[1]], ssems[b])

            def fetch_idx(base, b):
                pltpu.sync_copy(e_hbm.at[:, pl.ds(base, _CH)], idxs[b])

            ebase = s * (full * _CH)

            @pl.loop(0, nrounds)
            def _(t):
                base = ebase + t * (_NB * _CH)
                for b in range(_NB):
                    @pl.when(t > 0)
                    def _():
                        s_copy(b).wait()
                    fetch_idx(base + b * _CH, b)
                    g_copy(b).start()
                for b in range(_NB):
                    g_copy(b).wait()
                    s_copy(b).start(add=True)

            for b in range(_NB):
                s_copy(b).wait()

            if rem_tiles:
                @pl.when(s < rem_tiles)
                def _():
                    fetch_idx(_NUM_TILES * full * _CH + s * _CH, 0)
                    pltpu.sync_copy(x_hbm.at[idxs[0].at[0]], rows[0])
                    pltpu.sync_copy(rows[0], agg_sh.at[idxs[0].at[1]], add=True)

            plsc.subcore_barrier()

            def writeout(r0, nr):
                pltpu.sync_copy(agg_sh.at[pl.ds(r0, nr)], rows_a.at[pl.ds(0, nr)])
                pltpu.sync_copy(rows_a.at[pl.ds(0, nr)], out_hbm.at[pl.ds(r0, nr)])

            my_stripes(writeout, rem_rows)

        @pl.when(c == 0)
        def _():
            run_graph(xr_hbm, er_hbm, outr_hbm)

        @pl.when(c == 1)
        def _():
            run_graph(xp_hbm, ep_hbm, outp_hbm)

    return sc_agg


def _make_tc(N, D, B, RB):
    nb = N // RB

    def body(x_r_ref, a_r_ref, id_r_ref, x_p_ref, a_p_ref, id_p_ref,
             w1_ref, b1_ref, w2_ref, b2_ref, eps_ref,
             aw0, ab0, aw1, ab1, aw2, ab2, aw3, ab3,
             cw0, cb0, cw1, cb1, cw2, cb2,
             out_a, out_q, feat_ref):
        i = pl.program_id(0)

        @pl.when(i == 0)
        def _():
            feat_ref[...] = jnp.zeros_like(feat_ref)

        w1 = w1_ref[...]
        b1 = b1_ref[...]
        w2 = w2_ref[...]
        b2 = b2_ref[...]
        eps1 = 1.0 + eps_ref[0, 0]
        groups = ((x_r_ref, a_r_ref, id_r_ref), (x_p_ref, a_p_ref, id_p_ref))
        for g, (x_ref, a_ref, id_ref) in enumerate(groups):
            h0 = eps1 * x_ref[...] + a_ref[...]
            h1 = jnp.maximum(
                jnp.dot(h0, w1, preferred_element_type=jnp.float32) + b1, 0.0)
            h2 = jnp.dot(h1, w2, preferred_element_type=jnp.float32) + b2
            ids = id_ref[0]
            oh = (lax.broadcasted_iota(jnp.int32, (B, RB), 0) == ids
                  ).astype(jnp.bfloat16)
            feat_ref[g] += jnp.dot(oh, h2.astype(jnp.bfloat16),
                                   preferred_element_type=jnp.float32)

        @pl.when(i == nb - 1)
        def _():
            f = feat_ref[...]
            inp = jnp.concatenate([f[0], f[1]], axis=1)
            h = jnp.maximum(
                jnp.dot(inp, aw0[...], preferred_element_type=jnp.float32)
                + ab0[...], 0.0)
            h = jnp.maximum(
                jnp.dot(h, aw1[...], preferred_element_type=jnp.float32)
                + ab1[...], 0.0)
            h = jnp.maximum(
                jnp.dot(h, aw2[...], preferred_element_type=jnp.float32)
                + ab2[...], 0.0)
            out_a[...] = (jnp.dot(h, aw3[...], preferred_element_type=jnp.float32)
                          + ab3[...])
            hc = jnp.maximum(
                jnp.dot(inp, cw0[...], preferred_element_type=jnp.float32)
                + cb0[...], 0.0)
            hc = jnp.maximum(
                jnp.dot(hc, cw1[...], preferred_element_type=jnp.float32)
                + cb1[...], 0.0)
            out_q[...] = (jnp.dot(hc, cw2[...], preferred_element_type=jnp.float32)
                          + cb2[...])

    cst = lambda *shape: pl.BlockSpec(shape, lambda i: tuple(0 for _ in shape))
    blk = pl.BlockSpec((RB, D), lambda i: (i, 0))
    idb = pl.BlockSpec((1, 1, RB), lambda i: (i, 0, 0))

    def run(x_r, a_r, id_r, x_p, a_p, id_p, *ws):
        specs = ([blk, blk, idb, blk, blk, idb]
                 + [cst(*w.shape) for w in ws])
        return pl.pallas_call(
            body,
            grid=(nb,),
            in_specs=specs,
            out_specs=[cst(B, 2 * D), cst(B, 1)],
            out_shape=[
                jax.ShapeDtypeStruct((B, 2 * D), jnp.float32),
                jax.ShapeDtypeStruct((B, 1), jnp.float32),
            ],
            scratch_shapes=[pltpu.VMEM((2, B, D), jnp.float32)],
        )(x_r, a_r, id_r, x_p, a_p, id_p, *ws)

    return run


def kernel(reac_x, reac_edge_index, reac_graph_ids, prod_x, prod_edge_index,
           prod_graph_ids, gin_W1, gin_b1, gin_W2, gin_b2, gin_eps,
           aW0, ab0, aW1, ab1, aW2, ab2, aW3, ab3,
           cW0, cb0, cW1, cb1, cW2, cb2):
    N, D = reac_x.shape
    E = reac_edge_index.shape[1]
    B = 1024
    RB = 2000

    er = reac_edge_index.astype(jnp.int32)
    ep = prod_edge_index.astype(jnp.int32)
    agg_r, agg_p = _make_sc_agg(N, D, E)(reac_x, er, prod_x, ep)

    ids_r = reac_graph_ids.astype(jnp.int32).reshape(N // RB, 1, RB)
    ids_p = prod_graph_ids.astype(jnp.int32).reshape(N // RB, 1, RB)
    actions, q = _make_tc(N, D, B, RB)(
        reac_x, agg_r, ids_r, prod_x, agg_p, ids_p,
        gin_W1, gin_b1.reshape(1, D), gin_W2, gin_b2.reshape(1, D),
        gin_eps.reshape(1, 1),
        aW0, ab0.reshape(1, -1), aW1, ab1.reshape(1, -1),
        aW2, ab2.reshape(1, -1), aW3, ab3.reshape(1, -1),
        cW0, cb0.reshape(1, -1), cW1, cb1.reshape(1, -1),
        cW2, cb2.reshape(1, -1))
    return (actions, q)
